# baseline (speedup 1.0000x reference)
import functools

import jax
import jax.numpy as jnp
from jax import lax
from jax.experimental import pallas as pl
from jax.experimental.pallas import tpu as pltpu
from jax.experimental.pallas import tpu_sc as plsc

N_SETS = 2048
PHI_D = 64
CNT_W = 8
CHUNK = 128
NUM_CORES = 2
NUM_SUBCORES = 16
NW = NUM_CORES * NUM_SUBCORES
MAX_ALT = 10


def _lrelu(x):
    return jnp.maximum(x, 0.01 * x)


def _phi_body(x_ref, w0_ref, b0_ref, w1_ref, b1_ref, w2_ref, b2_ref, out_ref):
    bf = jnp.bfloat16
    h = jnp.dot(x_ref[...].astype(bf), w0_ref[...].astype(bf),
                preferred_element_type=jnp.float32)
    h = _lrelu(h + b0_ref[...])
    h = jnp.dot(h.astype(bf), w1_ref[...].astype(bf),
                preferred_element_type=jnp.float32)
    h = _lrelu(h + b1_ref[...])
    h = jnp.dot(h.astype(bf), w2_ref[...].astype(bf),
                preferred_element_type=jnp.float32)
    out_ref[...] = jax.nn.sigmoid(h + b2_ref[...])


def _phi_forward(x, w0, b0, w1, b1, w2, b2, row_off=0, n_rows=None,
                 block_rows=4096, interpret=False):
    n, d = x.shape
    if n_rows is None:
        n_rows = n
    off_blocks = row_off // block_rows
    grid = (n_rows // block_rows,)
    return pl.pallas_call(
        _phi_body,
        grid=grid,
        in_specs=[
            pl.BlockSpec((block_rows, d), lambda i: (i + off_blocks, 0)),
            pl.BlockSpec(w0.shape, lambda i: (0, 0)),
            pl.BlockSpec((1, w0.shape[1]), lambda i: (0, 0)),
            pl.BlockSpec(w1.shape, lambda i: (0, 0)),
            pl.BlockSpec((1, w1.shape[1]), lambda i: (0, 0)),
            pl.BlockSpec(w2.shape, lambda i: (0, 0)),
            pl.BlockSpec((1, w2.shape[1]), lambda i: (0, 0)),
        ],
        out_specs=pl.BlockSpec((block_rows, w2.shape[1]), lambda i: (i, 0)),
        out_shape=jax.ShapeDtypeStruct((n_rows, w2.shape[1]), jnp.float32),
        interpret=interpret,
    )(x, w0, b0.reshape(1, -1), w1, b1.reshape(1, -1), w2, b2.reshape(1, -1))


@functools.cache
def _make_segsum(n_rows):
    k_chunks = n_rows // NW // CHUNK
    mesh = plsc.VectorSubcoreMesh(
        core_axis_name="c", subcore_axis_name="s",
        num_cores=NUM_CORES, num_subcores=NUM_SUBCORES)
    sets_per_sub = N_SETS // NUM_SUBCORES

    @functools.partial(
        pl.kernel,
        out_type=[
            jax.ShapeDtypeStruct((NUM_CORES, N_SETS, PHI_D), jnp.float32),
            jax.ShapeDtypeStruct((NUM_CORES, N_SETS, CNT_W), jnp.float32),
        ],
        mesh=mesh,
        scratch_types=[
            pltpu.VMEM((CHUNK, PHI_D), jnp.float32),
            pltpu.VMEM((CHUNK, PHI_D), jnp.float32),
            pltpu.VMEM((CHUNK, CNT_W), jnp.float32),
            pltpu.VMEM((1, k_chunks, CHUNK), jnp.int32),
            pltpu.VMEM_SHARED((N_SETS, PHI_D), jnp.float32),
            pltpu.VMEM_SHARED((N_SETS, CNT_W), jnp.float32),
            pltpu.SemaphoreType.DMA,
            pltpu.SemaphoreType.DMA,
        ],
    )
    def segsum(vals_hbm, ids_hbm, z64_hbm, c16_hbm,
               sums_out, cnts_out,
               data_v0, data_v1, ones_v, idx_v, acc_s, cnt_s, sem0, sem1):
        c = lax.axis_index("c")
        s = lax.axis_index("s")
        wid = c * NUM_SUBCORES + s
        base = wid * k_chunks * CHUNK
        bufs = (data_v0, data_v1)
        sems = (sem0, sem1)
        cp = pltpu.async_copy(vals_hbm.at[pl.ds(base, CHUNK)], data_v0, sem0)
        handles = [cp, None]
        pltpu.sync_copy(ids_hbm.at[pl.ds(wid, 1)], idx_v)
        sl = pl.ds(s * sets_per_sub, sets_per_sub)
        pltpu.sync_copy(z64_hbm, data_v1)
        pltpu.sync_copy(data_v1, acc_s.at[sl])
        pltpu.sync_copy(c16_hbm.at[0], ones_v)
        pltpu.sync_copy(ones_v, cnt_s.at[sl])
        plsc.subcore_barrier()
        pltpu.sync_copy(c16_hbm.at[1], ones_v)

        for j in range(k_chunks):
            if j + 1 < k_chunks:
                handles[(j + 1) % 2] = pltpu.async_copy(
                    vals_hbm.at[pl.ds(base + (j + 1) * CHUNK, CHUNK)],
                    bufs[(j + 1) % 2], sems[(j + 1) % 2])
            handles[j % 2].wait()
            pltpu.sync_copy(bufs[j % 2], acc_s.at[idx_v.at[0, j]], add=True)
            pltpu.sync_copy(ones_v, cnt_s.at[idx_v.at[0, j]], add=True)

        plsc.subcore_barrier()
        pltpu.sync_copy(acc_s.at[sl], data_v0)
        pltpu.sync_copy(data_v0, sums_out.at[c, sl])
        pltpu.sync_copy(cnt_s.at[sl], ones_v)
        pltpu.sync_copy(ones_v, cnts_out.at[c, sl])

    return segsum


def _tail_body(rs0_ref, rc0_ref, rs1_ref, rc1_ref, as_ref, ac_ref, info_ref,
               ow0_ref, ob0_ref, ow1_ref, ob1_ref,
               rw0_ref, rb0_ref, rw1_ref, rb1_ref, rw2_ref, rb2_ref,
               out_ref):
    ref_sums = (rs0_ref[0] + rs0_ref[1]) + (rs1_ref[0] + rs1_ref[1])
    alt_sums = as_ref[0] + as_ref[1]
    ref_cnt = (rc0_ref[0, :, :1] + rc0_ref[1, :, :1]
               + rc1_ref[0, :, :1] + rc1_ref[1, :, :1])
    alt_cnt = ac_ref[0, :, :1] + ac_ref[1, :, :1]
    ref_means = ref_sums / ref_cnt
    alt_means = alt_sums / alt_cnt
    h = jnp.dot(info_ref[...], ow0_ref[...], preferred_element_type=jnp.float32)
    h = _lrelu(h + ob0_ref[...])
    h = jnp.dot(h, ow1_ref[...], preferred_element_type=jnp.float32)
    omega = jax.nn.sigmoid(h + ob1_ref[...])
    x = jnp.concatenate([ref_means, alt_means, omega], axis=1)
    h = jnp.dot(x, rw0_ref[...], preferred_element_type=jnp.float32)
    h = _lrelu(h + rb0_ref[...])
    h = jnp.dot(h, rw1_ref[...], preferred_element_type=jnp.float32)
    h = _lrelu(h + rb1_ref[...])
    out = jnp.sum(h * rw2_ref[...].reshape(1, -1), axis=1) + rb2_ref[0, 0]
    conf = jnp.sqrt(jnp.minimum(alt_cnt[:, 0], float(MAX_ALT)))
    out_ref[...] = (out * conf).reshape(out_ref.shape)


def _tail(ref_p0, ref_c0, ref_p1, ref_c1, alt_p, alt_c, info,
          ow0, ob0, ow1, ob1, rw0, rb0, rw1, rb1, rw2, rb2, interpret=False):
    full = lambda a: pl.BlockSpec(a.shape, lambda: tuple(0 for _ in a.shape))
    args = (ref_p0, ref_c0, ref_p1, ref_c1, alt_p, alt_c, info,
            ow0, ob0.reshape(1, -1), ow1, ob1.reshape(1, -1),
            rw0, rb0.reshape(1, -1), rw1, rb1.reshape(1, -1),
            rw2, rb2.reshape(1, 1))
    out = pl.pallas_call(
        _tail_body,
        in_specs=[full(a) for a in args],
        out_specs=pl.BlockSpec((N_SETS, 1), lambda: (0, 0)),
        out_shape=jax.ShapeDtypeStruct((N_SETS, 1), jnp.float32),
        interpret=interpret,
    )(*args)
    return out.reshape(N_SETS)


def kernel(ref_reads, alt_reads, info, ref_segment_ids, alt_segment_ids,
           phi_W0, phi_b0, phi_W1, phi_b1, phi_W2, phi_b2,
           omega_W0, omega_b0, omega_W1, omega_b1,
           rho_W0, rho_b0, rho_W1, rho_b1, rho_W2, rho_b2):
    z64 = jnp.zeros((CHUNK, PHI_D), jnp.float32)
    c16 = jnp.stack([jnp.zeros((CHUNK, CNT_W), jnp.float32),
                     jnp.ones((CHUNK, CNT_W), jnp.float32)])
    n_ref = ref_reads.shape[0]
    phi_w = (phi_W0, phi_b0, phi_W1, phi_b1, phi_W2, phi_b2)
    ids3 = lambda ids, a, b: ids[a:b].reshape(NW, -1, CHUNK)
    r0 = 28672
    n_alt = alt_reads.shape[0]
    phi_alt = _phi_forward(alt_reads, *phi_w)
    alt_sums, alt_cnts = _make_segsum(n_alt)(
        phi_alt, ids3(alt_segment_ids, 0, n_alt), z64, c16)
    phi_r0 = _phi_forward(ref_reads, *phi_w, row_off=0, n_rows=r0)
    r0_sums, r0_cnts = _make_segsum(r0)(
        phi_r0, ids3(ref_segment_ids, 0, r0), z64, c16)
    phi_r1 = _phi_forward(ref_reads, *phi_w, row_off=r0, n_rows=n_ref - r0)
    r1_sums, r1_cnts = _make_segsum(n_ref - r0)(
        phi_r1, ids3(ref_segment_ids, r0, n_ref), z64, c16)
    return _tail(r0_sums, r0_cnts, r1_sums, r1_cnts, alt_sums, alt_cnts, info,
                 omega_W0, omega_b0, omega_W1, omega_b1,
                 rho_W0, rho_b0, rho_W1, rho_b1, rho_W2, rho_b2)


# speedup vs baseline: 1.0413x; 1.0413x over previous
import functools

import jax
import jax.numpy as jnp
from jax import lax
from jax.experimental import pallas as pl
from jax.experimental.pallas import tpu as pltpu
from jax.experimental.pallas import tpu_sc as plsc

N_SETS = 2048
PHI_D = 64
CNT_W = 8
CHUNK = 128
NUM_CORES = 2
NUM_SUBCORES = 16
NW = NUM_CORES * NUM_SUBCORES
MAX_ALT = 10


def _lrelu(x):
    return jnp.maximum(x, 0.01 * x)


def _phi_body(x_ref, w0_ref, b0_ref, w1_ref, b1_ref, w2_ref, b2_ref, out_ref):
    bf = jnp.bfloat16
    h = jnp.dot(x_ref[...].astype(bf), w0_ref[...].astype(bf),
                preferred_element_type=jnp.float32)
    h = _lrelu(h + b0_ref[...])
    h = jnp.dot(h.astype(bf), w1_ref[...].astype(bf),
                preferred_element_type=jnp.float32)
    h = _lrelu(h + b1_ref[...])
    h = jnp.dot(h.astype(bf), w2_ref[...].astype(bf),
                preferred_element_type=jnp.float32)
    out_ref[...] = jax.nn.sigmoid(h + b2_ref[...])


def _phi_forward(x, w0, b0, w1, b1, w2, b2, row_off=0, n_rows=None,
                 block_rows=8192, interpret=False):
    n, d = x.shape
    if n_rows is None:
        n_rows = n
    off_blocks = row_off // block_rows
    grid = (n_rows // block_rows,)
    return pl.pallas_call(
        _phi_body,
        grid=grid,
        in_specs=[
            pl.BlockSpec((block_rows, d), lambda i: (i + off_blocks, 0)),
            pl.BlockSpec(w0.shape, lambda i: (0, 0)),
            pl.BlockSpec((1, w0.shape[1]), lambda i: (0, 0)),
            pl.BlockSpec(w1.shape, lambda i: (0, 0)),
            pl.BlockSpec((1, w1.shape[1]), lambda i: (0, 0)),
            pl.BlockSpec(w2.shape, lambda i: (0, 0)),
            pl.BlockSpec((1, w2.shape[1]), lambda i: (0, 0)),
        ],
        out_specs=pl.BlockSpec((block_rows, w2.shape[1]), lambda i: (i, 0)),
        out_shape=jax.ShapeDtypeStruct((n_rows, w2.shape[1]), jnp.float32),
        interpret=interpret,
    )(x, w0, b0.reshape(1, -1), w1, b1.reshape(1, -1), w2, b2.reshape(1, -1))


@functools.cache
def _make_segsum(n_rows, row_off=0):
    k_chunks = n_rows // NW // CHUNK
    off_chunks = row_off // CHUNK
    mesh = plsc.VectorSubcoreMesh(
        core_axis_name="c", subcore_axis_name="s",
        num_cores=NUM_CORES, num_subcores=NUM_SUBCORES)
    sets_per_sub = N_SETS // NUM_SUBCORES

    @functools.partial(
        pl.kernel,
        out_type=[
            jax.ShapeDtypeStruct((NUM_CORES, N_SETS, PHI_D), jnp.float32),
            jax.ShapeDtypeStruct((NUM_CORES, N_SETS, CNT_W), jnp.float32),
        ],
        mesh=mesh,
        scratch_types=[
            pltpu.VMEM((CHUNK, PHI_D), jnp.float32),
            pltpu.VMEM((CHUNK, PHI_D), jnp.float32),
            pltpu.VMEM((CHUNK, CNT_W), jnp.float32),
            pltpu.VMEM((k_chunks, CHUNK), jnp.int32),
            pltpu.VMEM_SHARED((N_SETS, PHI_D), jnp.float32),
            pltpu.VMEM_SHARED((N_SETS, CNT_W), jnp.float32),
            pltpu.SemaphoreType.DMA,
            pltpu.SemaphoreType.DMA,
        ],
    )
    def segsum(vals_hbm, ids_hbm, z64_hbm, c16_hbm,
               sums_out, cnts_out,
               data_v0, data_v1, ones_v, idx_v, acc_s, cnt_s, sem0, sem1):
        c = lax.axis_index("c")
        s = lax.axis_index("s")
        wid = c * NUM_SUBCORES + s
        base = wid * k_chunks * CHUNK
        bufs = (data_v0, data_v1)
        sems = (sem0, sem1)
        cp = pltpu.async_copy(vals_hbm.at[pl.ds(base, CHUNK)], data_v0, sem0)
        handles = [cp, None]
        pltpu.sync_copy(
            ids_hbm.at[pl.ds(off_chunks + wid * k_chunks, k_chunks)], idx_v)
        sl = pl.ds(s * sets_per_sub, sets_per_sub)
        pltpu.sync_copy(z64_hbm, data_v1)
        pltpu.sync_copy(data_v1, acc_s.at[sl])
        pltpu.sync_copy(c16_hbm.at[0], ones_v)
        pltpu.sync_copy(ones_v, cnt_s.at[sl])
        plsc.subcore_barrier()
        pltpu.sync_copy(c16_hbm.at[1], ones_v)

        for j in range(k_chunks):
            if j + 1 < k_chunks:
                handles[(j + 1) % 2] = pltpu.async_copy(
                    vals_hbm.at[pl.ds(base + (j + 1) * CHUNK, CHUNK)],
                    bufs[(j + 1) % 2], sems[(j + 1) % 2])
            handles[j % 2].wait()
            pltpu.sync_copy(bufs[j % 2], acc_s.at[idx_v.at[j]], add=True)
            pltpu.sync_copy(ones_v, cnt_s.at[idx_v.at[j]], add=True)

        plsc.subcore_barrier()
        pltpu.sync_copy(acc_s.at[sl], data_v0)
        pltpu.sync_copy(data_v0, sums_out.at[c, sl])
        pltpu.sync_copy(cnt_s.at[sl], ones_v)
        pltpu.sync_copy(ones_v, cnts_out.at[c, sl])

    return segsum


def _tail_body(rs0_ref, rc0_ref, rs1_ref, rc1_ref, as_ref, ac_ref, info_ref,
               ow0_ref, ob0_ref, ow1_ref, ob1_ref,
               rw0_ref, rb0_ref, rw1_ref, rb1_ref, rw2_ref, rb2_ref,
               out_ref):
    ref_sums = (rs0_ref[0] + rs0_ref[1]) + (rs1_ref[0] + rs1_ref[1])
    alt_sums = as_ref[0] + as_ref[1]
    ref_cnt = (rc0_ref[0, :, :1] + rc0_ref[1, :, :1]
               + rc1_ref[0, :, :1] + rc1_ref[1, :, :1])
    alt_cnt = ac_ref[0, :, :1] + ac_ref[1, :, :1]
    ref_means = ref_sums / ref_cnt
    alt_means = alt_sums / alt_cnt
    h = jnp.dot(info_ref[...], ow0_ref[...], preferred_element_type=jnp.float32)
    h = _lrelu(h + ob0_ref[...])
    h = jnp.dot(h, ow1_ref[...], preferred_element_type=jnp.float32)
    omega = jax.nn.sigmoid(h + ob1_ref[...])
    x = jnp.concatenate([ref_means, alt_means, omega], axis=1)
    h = jnp.dot(x, rw0_ref[...], preferred_element_type=jnp.float32)
    h = _lrelu(h + rb0_ref[...])
    h = jnp.dot(h, rw1_ref[...], preferred_element_type=jnp.float32)
    h = _lrelu(h + rb1_ref[...])
    out = jnp.sum(h * rw2_ref[...].reshape(1, -1), axis=1) + rb2_ref[0, 0]
    conf = jnp.sqrt(jnp.minimum(alt_cnt[:, 0], float(MAX_ALT)))
    out_ref[...] = (out * conf).reshape(out_ref.shape)


def _tail(ref_p0, ref_c0, ref_p1, ref_c1, alt_p, alt_c, info,
          ow0, ob0, ow1, ob1, rw0, rb0, rw1, rb1, rw2, rb2, interpret=False):
    full = lambda a: pl.BlockSpec(a.shape, lambda: tuple(0 for _ in a.shape))
    args = (ref_p0, ref_c0, ref_p1, ref_c1, alt_p, alt_c, info,
            ow0, ob0.reshape(1, -1), ow1, ob1.reshape(1, -1),
            rw0, rb0.reshape(1, -1), rw1, rb1.reshape(1, -1),
            rw2, rb2.reshape(1, 1))
    out = pl.pallas_call(
        _tail_body,
        in_specs=[full(a) for a in args],
        out_specs=pl.BlockSpec((N_SETS, 1), lambda: (0, 0)),
        out_shape=jax.ShapeDtypeStruct((N_SETS, 1), jnp.float32),
        interpret=interpret,
    )(*args)
    return out.reshape(N_SETS)


def kernel(ref_reads, alt_reads, info, ref_segment_ids, alt_segment_ids,
           phi_W0, phi_b0, phi_W1, phi_b1, phi_W2, phi_b2,
           omega_W0, omega_b0, omega_W1, omega_b1,
           rho_W0, rho_b0, rho_W1, rho_b1, rho_W2, rho_b2):
    z64 = jnp.zeros((CHUNK, PHI_D), jnp.float32)
    c16 = jnp.stack([jnp.zeros((CHUNK, CNT_W), jnp.float32),
                     jnp.ones((CHUNK, CNT_W), jnp.float32)])
    n_ref = ref_reads.shape[0]
    half = n_ref // 2
    ref_ids2 = ref_segment_ids.reshape(-1, CHUNK)
    alt_ids2 = alt_segment_ids.reshape(-1, CHUNK)
    phi_w = (phi_W0, phi_b0, phi_W1, phi_b1, phi_W2, phi_b2)
    phi_alt = _phi_forward(alt_reads, *phi_w)
    alt_sums, alt_cnts = _make_segsum(alt_reads.shape[0])(
        phi_alt, alt_ids2, z64, c16)
    phi_r0 = _phi_forward(ref_reads, *phi_w, row_off=0, n_rows=half)
    r0_sums, r0_cnts = _make_segsum(half, 0)(phi_r0, ref_ids2, z64, c16)
    phi_r1 = _phi_forward(ref_reads, *phi_w, row_off=half, n_rows=half)
    r1_sums, r1_cnts = _make_segsum(half, half)(phi_r1, ref_ids2, z64, c16)
    return _tail(r0_sums, r0_cnts, r1_sums, r1_cnts, alt_sums, alt_cnts, info,
                 omega_W0, omega_b0, omega_W1, omega_b1,
                 rho_W0, rho_b0, rho_W1, rho_b1, rho_W2, rho_b2)


# speedup vs baseline: 1.1186x; 1.0743x over previous
import functools

import jax
import jax.numpy as jnp
from jax import lax
from jax.experimental import pallas as pl
from jax.experimental.pallas import tpu as pltpu
from jax.experimental.pallas import tpu_sc as plsc

N_SETS = 2048
PHI_D = 64
CNT_W = 8
CHUNK = 128
NUM_CORES = 2
NUM_SUBCORES = 16
NW = NUM_CORES * NUM_SUBCORES
MAX_ALT = 10


def _lrelu(x):
    return jnp.maximum(x, 0.01 * x)


def _phi_body(x_ref, w0_ref, b0_ref, w1_ref, b1_ref, w2_ref, b2_ref, out_ref):
    h = jnp.dot(x_ref[...], w0_ref[...], preferred_element_type=jnp.float32)
    h = _lrelu(h + b0_ref[...])
    h = jnp.dot(h, w1_ref[...], preferred_element_type=jnp.float32)
    h = _lrelu(h + b1_ref[...])
    h = jnp.dot(h, w2_ref[...], preferred_element_type=jnp.float32)
    out_ref[...] = jax.nn.sigmoid(h + b2_ref[...])


def _phi_forward(x, w0, b0, w1, b1, w2, b2, row_off=0, n_rows=None,
                 block_rows=8192, interpret=False):
    n, d = x.shape
    if n_rows is None:
        n_rows = n
    off_blocks = row_off // block_rows
    grid = (n_rows // block_rows,)
    return pl.pallas_call(
        _phi_body,
        grid=grid,
        in_specs=[
            pl.BlockSpec((block_rows, d), lambda i: (i + off_blocks, 0)),
            pl.BlockSpec(w0.shape, lambda i: (0, 0)),
            pl.BlockSpec((1, w0.shape[1]), lambda i: (0, 0)),
            pl.BlockSpec(w1.shape, lambda i: (0, 0)),
            pl.BlockSpec((1, w1.shape[1]), lambda i: (0, 0)),
            pl.BlockSpec(w2.shape, lambda i: (0, 0)),
            pl.BlockSpec((1, w2.shape[1]), lambda i: (0, 0)),
        ],
        out_specs=pl.BlockSpec((block_rows, w2.shape[1]), lambda i: (i, 0)),
        out_shape=jax.ShapeDtypeStruct((n_rows, w2.shape[1]), jnp.float32),
        interpret=interpret,
    )(x, w0, b0.reshape(1, -1), w1, b1.reshape(1, -1), w2, b2.reshape(1, -1))


@functools.cache
def _make_segsum(n_rows, row_off=0):
    k_chunks = n_rows // NW // CHUNK
    off_chunks = row_off // CHUNK
    mesh = plsc.VectorSubcoreMesh(
        core_axis_name="c", subcore_axis_name="s",
        num_cores=NUM_CORES, num_subcores=NUM_SUBCORES)
    sets_per_sub = N_SETS // NUM_SUBCORES

    @functools.partial(
        pl.kernel,
        out_type=[
            jax.ShapeDtypeStruct((NUM_CORES, N_SETS, PHI_D), jnp.float32),
            jax.ShapeDtypeStruct((NUM_CORES, N_SETS, CNT_W), jnp.float32),
        ],
        mesh=mesh,
        scratch_types=[
            pltpu.VMEM((CHUNK, PHI_D), jnp.float32),
            pltpu.VMEM((CHUNK, PHI_D), jnp.float32),
            pltpu.VMEM((CHUNK, CNT_W), jnp.float32),
            pltpu.VMEM((k_chunks, CHUNK), jnp.int32),
            pltpu.VMEM_SHARED((N_SETS, PHI_D), jnp.float32),
            pltpu.VMEM_SHARED((N_SETS, CNT_W), jnp.float32),
            pltpu.SemaphoreType.DMA,
            pltpu.SemaphoreType.DMA,
        ],
    )
    def segsum(vals_hbm, ids_hbm, z64_hbm, c16_hbm,
               sums_out, cnts_out,
               data_v0, data_v1, ones_v, idx_v, acc_s, cnt_s, sem0, sem1):
        c = lax.axis_index("c")
        s = lax.axis_index("s")
        wid = c * NUM_SUBCORES + s
        base = wid * k_chunks * CHUNK
        bufs = (data_v0, data_v1)
        sems = (sem0, sem1)
        cp = pltpu.async_copy(vals_hbm.at[pl.ds(base, CHUNK)], data_v0, sem0)
        handles = [cp, None]
        pltpu.sync_copy(
            ids_hbm.at[pl.ds(off_chunks + wid * k_chunks, k_chunks)], idx_v)
        sl = pl.ds(s * sets_per_sub, sets_per_sub)
        pltpu.sync_copy(z64_hbm, data_v1)
        pltpu.sync_copy(data_v1, acc_s.at[sl])
        pltpu.sync_copy(c16_hbm.at[0], ones_v)
        pltpu.sync_copy(ones_v, cnt_s.at[sl])
        plsc.subcore_barrier()
        pltpu.sync_copy(c16_hbm.at[1], ones_v)

        for j in range(k_chunks):
            if j + 1 < k_chunks:
                handles[(j + 1) % 2] = pltpu.async_copy(
                    vals_hbm.at[pl.ds(base + (j + 1) * CHUNK, CHUNK)],
                    bufs[(j + 1) % 2], sems[(j + 1) % 2])
            handles[j % 2].wait()
            pltpu.sync_copy(bufs[j % 2], acc_s.at[idx_v.at[j]], add=True)
            pltpu.sync_copy(ones_v, cnt_s.at[idx_v.at[j]], add=True)

        plsc.subcore_barrier()
        pltpu.sync_copy(acc_s.at[sl], data_v0)
        pltpu.sync_copy(data_v0, sums_out.at[c, sl])
        pltpu.sync_copy(cnt_s.at[sl], ones_v)
        pltpu.sync_copy(ones_v, cnts_out.at[c, sl])

    return segsum


def _tail_body(rs_ref, rc_ref, as_ref, ac_ref, info_ref,
               ow0_ref, ob0_ref, ow1_ref, ob1_ref,
               rw0_ref, rb0_ref, rw1_ref, rb1_ref, rw2_ref, rb2_ref,
               out_ref):
    ref_sums = rs_ref[0] + rs_ref[1]
    alt_sums = as_ref[0] + as_ref[1]
    ref_cnt = rc_ref[0, :, :1] + rc_ref[1, :, :1]
    alt_cnt = ac_ref[0, :, :1] + ac_ref[1, :, :1]
    ref_means = ref_sums / ref_cnt
    alt_means = alt_sums / alt_cnt
    h = jnp.dot(info_ref[...], ow0_ref[...], preferred_element_type=jnp.float32)
    h = _lrelu(h + ob0_ref[...])
    h = jnp.dot(h, ow1_ref[...], preferred_element_type=jnp.float32)
    omega = jax.nn.sigmoid(h + ob1_ref[...])
    x = jnp.concatenate([ref_means, alt_means, omega], axis=1)
    h = jnp.dot(x, rw0_ref[...], preferred_element_type=jnp.float32)
    h = _lrelu(h + rb0_ref[...])
    h = jnp.dot(h, rw1_ref[...], preferred_element_type=jnp.float32)
    h = _lrelu(h + rb1_ref[...])
    out = jnp.sum(h * rw2_ref[...].reshape(1, -1), axis=1) + rb2_ref[0, 0]
    conf = jnp.sqrt(jnp.minimum(alt_cnt[:, 0], float(MAX_ALT)))
    out_ref[...] = (out * conf).reshape(out_ref.shape)


def _tail(ref_p, ref_c, alt_p, alt_c, info,
          ow0, ob0, ow1, ob1, rw0, rb0, rw1, rb1, rw2, rb2, interpret=False):
    full = lambda a: pl.BlockSpec(a.shape, lambda: tuple(0 for _ in a.shape))
    args = (ref_p, ref_c, alt_p, alt_c, info,
            ow0, ob0.reshape(1, -1), ow1, ob1.reshape(1, -1),
            rw0, rb0.reshape(1, -1), rw1, rb1.reshape(1, -1),
            rw2, rb2.reshape(1, 1))
    out = pl.pallas_call(
        _tail_body,
        in_specs=[full(a) for a in args],
        out_specs=pl.BlockSpec((N_SETS, 1), lambda: (0, 0)),
        out_shape=jax.ShapeDtypeStruct((N_SETS, 1), jnp.float32),
        interpret=interpret,
    )(*args)
    return out.reshape(N_SETS)


def kernel(ref_reads, alt_reads, info, ref_segment_ids, alt_segment_ids,
           phi_W0, phi_b0, phi_W1, phi_b1, phi_W2, phi_b2,
           omega_W0, omega_b0, omega_W1, omega_b1,
           rho_W0, rho_b0, rho_W1, rho_b1, rho_W2, rho_b2):
    z64 = jnp.zeros((CHUNK, PHI_D), jnp.float32)
    c16 = jnp.stack([jnp.zeros((CHUNK, CNT_W), jnp.float32),
                     jnp.ones((CHUNK, CNT_W), jnp.float32)])
    n_ref = ref_reads.shape[0]
    ref_ids2 = ref_segment_ids.reshape(-1, CHUNK)
    alt_ids2 = alt_segment_ids.reshape(-1, CHUNK)
    phi_w = (phi_W0, phi_b0, phi_W1, phi_b1, phi_W2, phi_b2)
    phi_alt = _phi_forward(alt_reads, *phi_w)
    alt_sums, alt_cnts = _make_segsum(alt_reads.shape[0])(
        phi_alt, alt_ids2, z64, c16)
    phi_ref = _phi_forward(ref_reads, *phi_w)
    ref_sums, ref_cnts = _make_segsum(n_ref)(phi_ref, ref_ids2, z64, c16)
    return _tail(ref_sums, ref_cnts, alt_sums, alt_cnts, info,
                 omega_W0, omega_b0, omega_W1, omega_b1,
                 rho_W0, rho_b0, rho_W1, rho_b1, rho_W2, rho_b2)
